# Initial kernel scaffold; baseline (speedup 1.0000x reference)
import functools

import jax
import jax.numpy as jnp
from jax import lax
from jax.experimental import pallas as pl
from jax.experimental.pallas import tpu as pltpu
from jax.experimental.pallas import tpu_sc as plsc

NC = 2
NS = 16
CL = 16


def _pick_chunk(ept: int) -> int:
    for b in range(128, 7, -8):
        if ept % b == 0:
            return b
    raise ValueError(f"per-tile edge count {ept} has no mult-of-8 divisor <= 128")


def _sc_aggregate(src, dst, x):
    n, d = x.shape
    e = src.shape[0]
    assert e % (NC * NS) == 0 and n % NS == 0 and d % CL == 0
    ept = e // (NC * NS)
    b = _pick_chunk(ept)
    nchunk = ept // b
    rpt = n // NS
    zr = min(rpt, 128)
    assert rpt % zr == 0

    mesh = plsc.VectorSubcoreMesh(core_axis_name="c", subcore_axis_name="s")

    @functools.partial(
        pl.kernel,
        out_type=(
            jax.ShapeDtypeStruct((NC, n, d), jnp.float32),
            jax.ShapeDtypeStruct((NC, n, CL), jnp.float32),
        ),
        mesh=mesh,
        scratch_types=[
            pltpu.VMEM_SHARED((n, d), jnp.float32),
            pltpu.VMEM_SHARED((n, CL), jnp.float32),
            pltpu.VMEM((1, b), jnp.int32),
            pltpu.VMEM((1, b), jnp.int32),
            pltpu.VMEM((b, d), jnp.float32),
            pltpu.VMEM((b, CL), jnp.float32),
            pltpu.VMEM((128, d), jnp.float32),
            pltpu.VMEM((n // NS, CL), jnp.float32),
            pltpu.SemaphoreType.DMA,
        ],
    )
    def sc_agg(src_hbm, dst_hbm, x_hbm, sum_out, cnt_out,
               sum_sh, cnt_sh, sidx, didx, rows, ones, zsum, zcnt, sem):
        c = lax.axis_index("c")
        s = lax.axis_index("s")
        zr_ = zsum.shape[0]

        @pl.loop(0, zr_)
        def _(r):
            @pl.loop(0, d // CL)
            def _(k):
                zsum[r, pl.ds(k * CL, CL)] = jnp.zeros((CL,), jnp.float32)

        @pl.loop(0, rpt)
        def _(r):
            zcnt[r, :] = jnp.zeros((CL,), jnp.float32)

        @pl.loop(0, b)
        def _(r):
            ones[r, :] = jnp.ones((CL,), jnp.float32)

        row0 = s * rpt

        @pl.loop(0, rpt // zr_)
        def _(i):
            pltpu.sync_copy(zsum, sum_sh.at[pl.ds(row0 + i * zr_, zr_)])

        pltpu.sync_copy(zcnt, cnt_sh.at[pl.ds(row0, rpt)])
        plsc.subcore_barrier()

        tile_base = (c * NS + s) * ept

        @pl.loop(0, nchunk)
        def _(g):
            base = tile_base + g * b
            pltpu.sync_copy(src_hbm.at[pl.ds(base, b)], sidx.at[0])
            pltpu.sync_copy(dst_hbm.at[pl.ds(base, b)], didx.at[0])
            pltpu.async_copy(x_hbm.at[sidx.at[0]], rows, sem).wait()
            pltpu.sync_copy(rows, sum_sh.at[didx.at[0]], add=True)
            pltpu.sync_copy(ones, cnt_sh.at[didx.at[0]], add=True)

        plsc.subcore_barrier()
        pltpu.sync_copy(sum_sh.at[pl.ds(row0, rpt)],
                        sum_out.at[c].at[pl.ds(row0, rpt)])
        pltpu.sync_copy(cnt_sh.at[pl.ds(row0, rpt)],
                        cnt_out.at[c].at[pl.ds(row0, rpt)])

    return sc_agg(src, dst, x)


def _finish_body(sum_ref, cnt_ref, x_ref, wl_ref, wr_ref, bl_ref, o_ref):
    s = sum_ref[0] + sum_ref[1]
    cnt = cnt_ref[0, :, 0:1] + cnt_ref[1, :, 0:1]
    mean = s / jnp.maximum(cnt, 1.0)
    acc = jnp.dot(mean, wl_ref[...], preferred_element_type=jnp.float32)
    acc = acc + jnp.dot(x_ref[...], wr_ref[...], preferred_element_type=jnp.float32)
    acc = acc + bl_ref[...]
    o_ref[...] = jnp.maximum(acc, 0.0)


def _tc_finish(x, sum_p, cnt_p, wl_t, wr_t, bl):
    n, d = x.shape
    bt = 2000 if n % 2000 == 0 else n
    grid = (n // bt,)
    return pl.pallas_call(
        _finish_body,
        grid=grid,
        in_specs=[
            pl.BlockSpec((NC, bt, d), lambda i: (0, i, 0)),
            pl.BlockSpec((NC, bt, CL), lambda i: (0, i, 0)),
            pl.BlockSpec((bt, d), lambda i: (i, 0)),
            pl.BlockSpec((d, d), lambda i: (0, 0)),
            pl.BlockSpec((d, d), lambda i: (0, 0)),
            pl.BlockSpec((1, d), lambda i: (0, 0)),
        ],
        out_specs=pl.BlockSpec((bt, d), lambda i: (i, 0)),
        out_shape=jax.ShapeDtypeStruct((n, d), jnp.float32),
    )(sum_p, cnt_p, x, wl_t, wr_t, bl)


def kernel(x, edge_index, W_l, b_l, W_r):
    src = edge_index[0].astype(jnp.int32)
    dst = edge_index[1].astype(jnp.int32)
    sum_p, cnt_p = _sc_aggregate(src, dst, x)
    return _tc_finish(x, sum_p, cnt_p, W_l.T, W_r.T, b_l.reshape(1, -1))


# speedup vs baseline: 6.2336x; 6.2336x over previous
import dataclasses
import functools

import jax
import jax.numpy as jnp
from jax import lax
from jax.experimental import pallas as pl
from jax.experimental.pallas import tpu as pltpu
from jax.experimental.pallas import tpu_sc as plsc

NC = 2
NS = 16
CL = 16


def _pick_chunk(ept: int) -> int:
    for b in range(128, 15, -16):
        if ept % b == 0:
            return b
    raise ValueError(f"per-tile edge count {ept} has no mult-of-16 divisor <= 128")


def _sc_aggregate(src, dst, x):
    n, d = x.shape
    e = src.shape[0]
    assert e % (NC * NS) == 0 and d % CL == 0
    ept = e // (NC * NS)
    b = _pick_chunk(ept)
    nchunk = ept // b
    npad = -(-n // (NS * CL)) * NS * CL
    rpt = npad // NS
    assert rpt % b == 0 and rpt % CL == 0

    mesh = plsc.VectorSubcoreMesh(core_axis_name="c", subcore_axis_name="s")
    cp = pltpu.CompilerParams()
    if "needs_layout_passes" in pltpu.CompilerParams.__dataclass_fields__:
        cp = dataclasses.replace(cp, needs_layout_passes=False)

    @functools.partial(
        pl.kernel,
        compiler_params=cp,
        out_type=(
            jax.ShapeDtypeStruct((NC, npad, d), jnp.float32),
            jax.ShapeDtypeStruct((NC, npad), jnp.float32),
        ),
        mesh=mesh,
        scratch_types=[
            pltpu.VMEM_SHARED((npad, d), jnp.float32),
            pltpu.VMEM_SHARED((NS, NS, rpt), jnp.float32),
            pltpu.VMEM((1, b), jnp.int32),
            pltpu.VMEM((1, b), jnp.int32),
            pltpu.VMEM((b, d), jnp.float32),
            pltpu.VMEM((npad,), jnp.float32),
            pltpu.VMEM((rpt,), jnp.float32),
            pltpu.SemaphoreType.DMA,
        ],
    )
    def sc_agg(src_hbm, dst_hbm, x_hbm, sum_out, cnt_out,
               sum_sh, cnt_st, sidx, didx, rows, hist, red, sem):
        c = lax.axis_index("c")
        s = lax.axis_index("s")

        @pl.loop(0, b)
        def _(r):
            @pl.loop(0, d // CL)
            def _(k):
                rows[r, pl.ds(k * CL, CL)] = jnp.zeros((CL,), jnp.float32)

        @pl.loop(0, npad // CL)
        def _(i):
            hist[pl.ds(i * CL, CL)] = jnp.zeros((CL,), jnp.float32)

        row0 = s * rpt

        @pl.loop(0, rpt // b)
        def _(i):
            pltpu.sync_copy(rows, sum_sh.at[pl.ds(row0 + i * b, b)])

        plsc.subcore_barrier()

        tile_base = (c * NS + s) * ept

        @pl.loop(0, nchunk)
        def _(g):
            base = tile_base + g * b
            pltpu.sync_copy(src_hbm.at[pl.ds(base, b)], sidx.at[0])
            pltpu.sync_copy(dst_hbm.at[pl.ds(base, b)], didx.at[0])

            @pl.loop(0, b // CL)
            def _(k):
                dvec = didx[0, pl.ds(k * CL, CL)]
                run, last = plsc.scan_count(dvec)
                plsc.addupdate_scatter(hist, [dvec],
                                       run.astype(jnp.float32), mask=last)

            pltpu.async_copy(x_hbm.at[sidx.at[0]], rows, sem).wait()
            pltpu.sync_copy(rows, sum_sh.at[didx.at[0]], add=True)

        @pl.loop(0, NS)
        def _(t):
            pltpu.sync_copy(hist.at[pl.ds(t * rpt, rpt)], cnt_st.at[t, s])

        plsc.subcore_barrier()

        @pl.loop(0, NS)
        def _(r):
            pltpu.sync_copy(cnt_st.at[s, r], hist.at[pl.ds(r * rpt, rpt)])

        @pl.loop(0, rpt // CL)
        def _(j):
            total = hist[pl.ds(j * CL, CL)]
            for r in range(1, NS):
                total = total + hist[pl.ds(r * rpt + j * CL, CL)]
            red[pl.ds(j * CL, CL)] = total

        pltpu.sync_copy(sum_sh.at[pl.ds(row0, rpt)],
                        sum_out.at[c].at[pl.ds(row0, rpt)])
        pltpu.sync_copy(red, cnt_out.at[c].at[pl.ds(row0, rpt)])

    return sc_agg(src, dst, x)


def _finish_body(sum_ref, inv_ref, x_ref, wl_ref, wr_ref, bl_ref, o_ref):
    mean = (sum_ref[0] + sum_ref[1]) * inv_ref[...]
    acc = jnp.dot(mean, wl_ref[...], preferred_element_type=jnp.float32)
    acc = acc + jnp.dot(x_ref[...], wr_ref[...], preferred_element_type=jnp.float32)
    acc = acc + bl_ref[...]
    o_ref[...] = jnp.maximum(acc, 0.0)


def _tc_finish(x, sum_p, inv_cnt, wl_t, wr_t, bl):
    n, d = x.shape
    bt = 2000 if n % 2000 == 0 else n
    grid = (n // bt,)
    return pl.pallas_call(
        _finish_body,
        grid=grid,
        in_specs=[
            pl.BlockSpec((NC, bt, d), lambda i: (0, i, 0)),
            pl.BlockSpec((bt, 1), lambda i: (i, 0)),
            pl.BlockSpec((bt, d), lambda i: (i, 0)),
            pl.BlockSpec((d, d), lambda i: (0, 0)),
            pl.BlockSpec((d, d), lambda i: (0, 0)),
            pl.BlockSpec((1, d), lambda i: (0, 0)),
        ],
        out_specs=pl.BlockSpec((bt, d), lambda i: (i, 0)),
        out_shape=jax.ShapeDtypeStruct((n, d), jnp.float32),
    )(sum_p, inv_cnt, x, wl_t, wr_t, bl)


def kernel(x, edge_index, W_l, b_l, W_r):
    n = x.shape[0]
    src = edge_index[0].astype(jnp.int32)
    dst = edge_index[1].astype(jnp.int32)
    sum_p, cnt_p = _sc_aggregate(src, dst, x)
    cnt = (cnt_p[0] + cnt_p[1])[:n]
    inv_cnt = (1.0 / jnp.maximum(cnt, 1.0)).reshape(n, 1)
    return _tc_finish(x, sum_p, inv_cnt, W_l.T, W_r.T, b_l.reshape(1, -1))


# speedup vs baseline: 13.4965x; 2.1651x over previous
import dataclasses
import functools

import jax
import jax.numpy as jnp
from jax import lax
from jax.experimental import pallas as pl
from jax.experimental.pallas import tpu as pltpu
from jax.experimental.pallas import tpu_sc as plsc

NC = 2
NS = 16
CL = 16


def _pick_chunk(ept: int) -> int:
    for b in range(128, 15, -16):
        if ept % b == 0:
            return b
    raise ValueError(f"per-tile edge count {ept} has no mult-of-16 divisor <= 128")


def _sc_aggregate(src, dst, x):
    n, d = x.shape
    e = src.shape[0]
    assert e % (NC * NS) == 0 and d % CL == 0
    ept = e // (NC * NS)
    b = _pick_chunk(ept)
    nchunk = ept // b
    npad = -(-n // (NS * CL)) * NS * CL
    rpt = npad // NS
    assert rpt % b == 0 and rpt % CL == 0

    mesh = plsc.VectorSubcoreMesh(core_axis_name="c", subcore_axis_name="s")
    cp = pltpu.CompilerParams()
    if "needs_layout_passes" in pltpu.CompilerParams.__dataclass_fields__:
        cp = dataclasses.replace(cp, needs_layout_passes=False)

    @functools.partial(
        pl.kernel,
        compiler_params=cp,
        out_type=(
            jax.ShapeDtypeStruct((NC, npad, d), jnp.float32),
            jax.ShapeDtypeStruct((NC, NS, npad), jnp.float32),
        ),
        mesh=mesh,
        scratch_types=[
            pltpu.VMEM_SHARED((npad, d), jnp.float32),
            pltpu.VMEM((4, 1, b), jnp.int32),
            pltpu.VMEM((4, 1, b), jnp.int32),
            pltpu.VMEM((b, d), jnp.float32),
            pltpu.VMEM((b, d), jnp.float32),
            pltpu.VMEM((npad,), jnp.float32),
            pltpu.SemaphoreType.DMA,
            pltpu.SemaphoreType.DMA,
            pltpu.SemaphoreType.DMA,
            pltpu.SemaphoreType.DMA,
            pltpu.SemaphoreType.DMA,
            pltpu.SemaphoreType.DMA,
        ],
    )
    def sc_agg(src_hbm, dst_hbm, x_hbm, sum_out, cnt_out,
               sum_sh, sidx, didx, rows0, rows1, hist,
               gsem0, gsem1, isem0, isem1, isem2, isem3):
        c = lax.axis_index("c")
        s = lax.axis_index("s")
        rows = (rows0, rows1)
        gsem = (gsem0, gsem1)
        isem = (isem0, isem1, isem2, isem3)

        @pl.loop(0, b)
        def _(r):
            @pl.loop(0, d // CL)
            def _(k):
                rows0[r, pl.ds(k * CL, CL)] = jnp.zeros((CL,), jnp.float32)

        @pl.loop(0, npad // CL)
        def _(i):
            hist[pl.ds(i * CL, CL)] = jnp.zeros((CL,), jnp.float32)

        row0 = s * rpt

        @pl.loop(0, rpt // b)
        def _(i):
            pltpu.sync_copy(rows0, sum_sh.at[pl.ds(row0 + i * b, b)])

        plsc.subcore_barrier()

        tile_base = (c * NS + s) * ept

        def load_idx(slot, g, sem):
            base = tile_base + g * b
            pltpu.async_copy(src_hbm.at[pl.ds(base, b)], sidx.at[slot, 0], sem)
            pltpu.async_copy(dst_hbm.at[pl.ds(base, b)], didx.at[slot, 0], sem)

        def wait_idx(slot, sem):
            pltpu.make_async_copy(src_hbm.at[pl.ds(0, b)],
                                  sidx.at[slot, 0], sem).wait()
            pltpu.make_async_copy(dst_hbm.at[pl.ds(0, b)],
                                  didx.at[slot, 0], sem).wait()

        load_idx(0, 0, isem[0])
        wait_idx(0, isem[0])
        load_idx(1, 1, isem[1])
        wait_idx(1, isem[1])
        pltpu.async_copy(x_hbm.at[sidx.at[0, 0]], rows0, gsem0)
        pltpu.async_copy(x_hbm.at[sidx.at[1, 0]], rows1, gsem1)
        load_idx(2, 2, isem[2])

        @pl.loop(0, (nchunk + 3) // 4)
        def _(t):
            for r in range(4):
                g = t * 4 + r
                p = r % 2

                @pl.when(g < nchunk)
                def _(g=g, p=p, r=r):
                    pltpu.make_async_copy(x_hbm.at[sidx.at[r, 0]],
                                          rows[p], gsem[p]).wait()

                    @pl.loop(0, b // CL)
                    def _(k):
                        dvec = didx[r, 0, pl.ds(k * CL, CL)]
                        run, last = plsc.scan_count(dvec)
                        plsc.addupdate_scatter(hist, [dvec],
                                               run.astype(jnp.float32),
                                               mask=last)

                    pltpu.sync_copy(rows[p], sum_sh.at[didx.at[r, 0]],
                                    add=True)

                    @pl.when(g + 3 < nchunk)
                    def _(g=g, r=r):
                        load_idx((r + 3) % 4, g + 3, isem[(r + 3) % 4])

                    @pl.when(g + 2 < nchunk)
                    def _(g=g, p=p, r=r):
                        wait_idx((r + 2) % 4, isem[(r + 2) % 4])
                        pltpu.async_copy(x_hbm.at[sidx.at[(r + 2) % 4, 0]],
                                         rows[p], gsem[p])

        pltpu.sync_copy(hist, cnt_out.at[c].at[s])

        plsc.subcore_barrier()
        pltpu.sync_copy(sum_sh.at[pl.ds(row0, rpt)],
                        sum_out.at[c].at[pl.ds(row0, rpt)])

    return sc_agg(src, dst, x)


def _finish_body(sum_ref, inv_ref, x_ref, wl_ref, wr_ref, bl_ref, o_ref):
    mean = (sum_ref[0] + sum_ref[1]) * inv_ref[...]
    acc = jnp.dot(mean, wl_ref[...], preferred_element_type=jnp.float32)
    acc = acc + jnp.dot(x_ref[...], wr_ref[...], preferred_element_type=jnp.float32)
    acc = acc + bl_ref[...]
    o_ref[...] = jnp.maximum(acc, 0.0)


def _tc_finish(x, sum_p, inv_cnt, wl_t, wr_t, bl):
    n, d = x.shape
    bt = 2000 if n % 2000 == 0 else n
    grid = (n // bt,)
    return pl.pallas_call(
        _finish_body,
        grid=grid,
        in_specs=[
            pl.BlockSpec((NC, bt, d), lambda i: (0, i, 0)),
            pl.BlockSpec((bt, 1), lambda i: (i, 0)),
            pl.BlockSpec((bt, d), lambda i: (i, 0)),
            pl.BlockSpec((d, d), lambda i: (0, 0)),
            pl.BlockSpec((d, d), lambda i: (0, 0)),
            pl.BlockSpec((1, d), lambda i: (0, 0)),
        ],
        out_specs=pl.BlockSpec((bt, d), lambda i: (i, 0)),
        out_shape=jax.ShapeDtypeStruct((n, d), jnp.float32),
    )(sum_p, inv_cnt, x, wl_t, wr_t, bl)


def kernel(x, edge_index, W_l, b_l, W_r):
    n = x.shape[0]
    src = edge_index[0].astype(jnp.int32)
    dst = edge_index[1].astype(jnp.int32)
    sum_p, cnt_p = _sc_aggregate(src, dst, x)
    cnt = jnp.sum(cnt_p, axis=(0, 1))[:n]
    inv_cnt = (1.0 / jnp.maximum(cnt, 1.0)).reshape(n, 1)
    return _tc_finish(x, sum_p, inv_cnt, W_l.T, W_r.T, b_l.reshape(1, -1))


# speedup vs baseline: 13.7444x; 1.0184x over previous
import dataclasses
import functools

import jax
import jax.numpy as jnp
from jax import lax
from jax.experimental import pallas as pl
from jax.experimental.pallas import tpu as pltpu
from jax.experimental.pallas import tpu_sc as plsc

NC = 2
NS = 16
CL = 16


def _pick_chunk(ept: int) -> int:
    for b in range(128, 15, -16):
        if ept % b == 0:
            return b
    raise ValueError(f"per-tile edge count {ept} has no mult-of-16 divisor <= 128")


def _sc_aggregate(src, dst, x):
    n, d = x.shape
    e = src.shape[0]
    assert e % (NC * NS) == 0 and d % CL == 0
    ept = e // (NC * NS)
    b = _pick_chunk(ept)
    nchunk = ept // b
    npad = -(-n // (NS * CL)) * NS * CL
    rpt = npad // NS
    assert rpt % b == 0 and rpt % CL == 0

    mesh = plsc.VectorSubcoreMesh(core_axis_name="c", subcore_axis_name="s")
    cp = pltpu.CompilerParams()
    if "needs_layout_passes" in pltpu.CompilerParams.__dataclass_fields__:
        cp = dataclasses.replace(cp, needs_layout_passes=False)

    @functools.partial(
        pl.kernel,
        compiler_params=cp,
        out_type=(
            jax.ShapeDtypeStruct((NC, npad, d), jnp.float32),
            jax.ShapeDtypeStruct((NC, NS, npad), jnp.float32),
        ),
        mesh=mesh,
        scratch_types=[
            pltpu.VMEM_SHARED((npad, d), jnp.float32),
            pltpu.VMEM((4, 1, b), jnp.int32),
            pltpu.VMEM((4, 1, b), jnp.int32),
            pltpu.VMEM((b, d), jnp.float32),
            pltpu.VMEM((b, d), jnp.float32),
            pltpu.VMEM((npad,), jnp.float32),
            pltpu.SemaphoreType.DMA,
            pltpu.SemaphoreType.DMA,
            pltpu.SemaphoreType.DMA,
            pltpu.SemaphoreType.DMA,
            pltpu.SemaphoreType.DMA,
            pltpu.SemaphoreType.DMA,
            pltpu.SemaphoreType.DMA,
            pltpu.SemaphoreType.DMA,
        ],
    )
    def sc_agg(src_hbm, dst_hbm, x_hbm, sum_out, cnt_out,
               sum_sh, sidx, didx, rows0, rows1, hist,
               gsem0, gsem1, ssem0, ssem1, isem0, isem1, isem2, isem3):
        c = lax.axis_index("c")
        s = lax.axis_index("s")
        rows = (rows0, rows1)
        gsem = (gsem0, gsem1)
        ssem = (ssem0, ssem1)
        isem = (isem0, isem1, isem2, isem3)

        @pl.loop(0, b)
        def _(r):
            @pl.loop(0, d // CL)
            def _(k):
                rows0[r, pl.ds(k * CL, CL)] = jnp.zeros((CL,), jnp.float32)

        @pl.loop(0, npad // CL)
        def _(i):
            hist[pl.ds(i * CL, CL)] = jnp.zeros((CL,), jnp.float32)

        row0 = s * rpt

        @pl.loop(0, rpt // b)
        def _(i):
            pltpu.sync_copy(rows0, sum_sh.at[pl.ds(row0 + i * b, b)])

        plsc.subcore_barrier()

        tile_base = (c * NS + s) * ept

        def load_idx(slot, g, sem):
            base = tile_base + g * b
            pltpu.async_copy(src_hbm.at[pl.ds(base, b)], sidx.at[slot, 0], sem)
            pltpu.async_copy(dst_hbm.at[pl.ds(base, b)], didx.at[slot, 0], sem)

        def wait_idx(slot, sem):
            pltpu.make_async_copy(src_hbm.at[pl.ds(0, b)],
                                  sidx.at[slot, 0], sem).wait()
            pltpu.make_async_copy(dst_hbm.at[pl.ds(0, b)],
                                  didx.at[slot, 0], sem).wait()

        load_idx(0, 0, isem[0])
        wait_idx(0, isem[0])
        load_idx(1, 1, isem[1])
        wait_idx(1, isem[1])
        pltpu.async_copy(x_hbm.at[sidx.at[0, 0]], rows0, gsem0)
        pltpu.async_copy(x_hbm.at[sidx.at[1, 0]], rows1, gsem1)
        load_idx(2, 2, isem[2])

        @pl.loop(0, (nchunk + 3) // 4)
        def _(t):
            for r in range(4):
                g = t * 4 + r
                p = r % 2

                @pl.when(g < nchunk)
                def _(g=g, p=p, r=r):
                    pltpu.make_async_copy(x_hbm.at[sidx.at[r, 0]],
                                          rows[p], gsem[p]).wait()

                    pltpu.async_copy(rows[p], sum_sh.at[didx.at[r, 0]],
                                     ssem[p], add=True)

                    @pl.loop(0, b // CL)
                    def _(k):
                        dvec = didx[r, 0, pl.ds(k * CL, CL)]
                        run, last = plsc.scan_count(dvec)
                        plsc.addupdate_scatter(hist, [dvec],
                                               run.astype(jnp.float32),
                                               mask=last)

                    @pl.when(g + 3 < nchunk)
                    def _(g=g, r=r):
                        load_idx((r + 3) % 4, g + 3, isem[(r + 3) % 4])

                    @pl.when(g + 2 < nchunk)
                    def _(g=g, r=r):
                        wait_idx((r + 2) % 4, isem[(r + 2) % 4])

                    pltpu.make_async_copy(rows[p],
                                          sum_sh.at[didx.at[r, 0]],
                                          ssem[p]).wait()

                    @pl.when(g + 2 < nchunk)
                    def _(g=g, p=p, r=r):
                        pltpu.async_copy(x_hbm.at[sidx.at[(r + 2) % 4, 0]],
                                         rows[p], gsem[p])

        pltpu.sync_copy(hist, cnt_out.at[c].at[s])

        plsc.subcore_barrier()
        pltpu.sync_copy(sum_sh.at[pl.ds(row0, rpt)],
                        sum_out.at[c].at[pl.ds(row0, rpt)])

    return sc_agg(src, dst, x)


def _finish_body(sum_ref, inv_ref, x_ref, wl_ref, wr_ref, bl_ref, o_ref):
    mean = (sum_ref[0] + sum_ref[1]) * inv_ref[...]
    acc = jnp.dot(mean, wl_ref[...], preferred_element_type=jnp.float32)
    acc = acc + jnp.dot(x_ref[...], wr_ref[...], preferred_element_type=jnp.float32)
    acc = acc + bl_ref[...]
    o_ref[...] = jnp.maximum(acc, 0.0)


def _tc_finish(x, sum_p, inv_cnt, wl_t, wr_t, bl):
    n, d = x.shape
    bt = 2000 if n % 2000 == 0 else n
    grid = (n // bt,)
    return pl.pallas_call(
        _finish_body,
        grid=grid,
        in_specs=[
            pl.BlockSpec((NC, bt, d), lambda i: (0, i, 0)),
            pl.BlockSpec((bt, 1), lambda i: (i, 0)),
            pl.BlockSpec((bt, d), lambda i: (i, 0)),
            pl.BlockSpec((d, d), lambda i: (0, 0)),
            pl.BlockSpec((d, d), lambda i: (0, 0)),
            pl.BlockSpec((1, d), lambda i: (0, 0)),
        ],
        out_specs=pl.BlockSpec((bt, d), lambda i: (i, 0)),
        out_shape=jax.ShapeDtypeStruct((n, d), jnp.float32),
    )(sum_p, inv_cnt, x, wl_t, wr_t, bl)


def kernel(x, edge_index, W_l, b_l, W_r):
    n = x.shape[0]
    src = edge_index[0].astype(jnp.int32)
    dst = edge_index[1].astype(jnp.int32)
    sum_p, cnt_p = _sc_aggregate(src, dst, x)
    cnt = jnp.sum(cnt_p, axis=(0, 1))[:n]
    inv_cnt = (1.0 / jnp.maximum(cnt, 1.0)).reshape(n, 1)
    return _tc_finish(x, sum_p, inv_cnt, W_l.T, W_r.T, b_l.reshape(1, -1))


# speedup vs baseline: 15.1722x; 1.1039x over previous
import dataclasses
import functools

import jax
import jax.numpy as jnp
from jax import lax
from jax.experimental import pallas as pl
from jax.experimental.pallas import tpu as pltpu
from jax.experimental.pallas import tpu_sc as plsc

NC = 2
NS = 16
CL = 16


def _pick_chunk(ept: int) -> int:
    for b in range(128, 15, -16):
        if ept % b == 0:
            return b
    raise ValueError(f"per-tile edge count {ept} has no mult-of-16 divisor <= 128")


def _sc_aggregate(src, dst, x):
    n, d = x.shape
    e = src.shape[0]
    assert e % (NC * NS) == 0 and d % CL == 0
    ept = e // (NC * NS)
    b = _pick_chunk(ept)
    nchunk = ept // b
    npad = -(-n // (NS * CL)) * NS * CL
    rpt = npad // NS
    assert rpt % b == 0 and rpt % CL == 0

    mesh = plsc.VectorSubcoreMesh(core_axis_name="c", subcore_axis_name="s")
    cp = pltpu.CompilerParams()
    if "needs_layout_passes" in pltpu.CompilerParams.__dataclass_fields__:
        cp = dataclasses.replace(cp, needs_layout_passes=False)

    @functools.partial(
        pl.kernel,
        compiler_params=cp,
        out_type=(
            jax.ShapeDtypeStruct((NC, npad, d), jnp.float32),
            jax.ShapeDtypeStruct((NC, NS, npad), jnp.float32),
        ),
        mesh=mesh,
        scratch_types=[
            pltpu.VMEM_SHARED((npad, d), jnp.float32),
            pltpu.VMEM((6, 1, b), jnp.int32),
            pltpu.VMEM((6, 1, b), jnp.int32),
            pltpu.VMEM((b, d), jnp.float32),
            pltpu.VMEM((b, d), jnp.float32),
            pltpu.VMEM((b, d), jnp.float32),
            pltpu.VMEM((npad,), jnp.float32),
            pltpu.SemaphoreType.DMA,
            pltpu.SemaphoreType.DMA,
            pltpu.SemaphoreType.DMA,
            pltpu.SemaphoreType.DMA,
            pltpu.SemaphoreType.DMA,
            pltpu.SemaphoreType.DMA,
            pltpu.SemaphoreType.DMA,
            pltpu.SemaphoreType.DMA,
            pltpu.SemaphoreType.DMA,
            pltpu.SemaphoreType.DMA,
            pltpu.SemaphoreType.DMA,
            pltpu.SemaphoreType.DMA,
        ],
    )
    def sc_agg(src_hbm, dst_hbm, x_hbm, sum_out, cnt_out,
               sum_sh, sidx, didx, rows0, rows1, rows2, hist,
               gsem0, gsem1, gsem2, ssem0, ssem1, ssem2,
               isem0, isem1, isem2, isem3, isem4, isem5):
        c = lax.axis_index("c")
        s = lax.axis_index("s")
        rows = (rows0, rows1, rows2)
        gsem = (gsem0, gsem1, gsem2)
        ssem = (ssem0, ssem1, ssem2)
        isem = (isem0, isem1, isem2, isem3, isem4, isem5)

        @pl.loop(0, b)
        def _(r):
            @pl.loop(0, d // CL)
            def _(k):
                rows0[r, pl.ds(k * CL, CL)] = jnp.zeros((CL,), jnp.float32)

        @pl.loop(0, npad // CL)
        def _(i):
            hist[pl.ds(i * CL, CL)] = jnp.zeros((CL,), jnp.float32)

        row0 = s * rpt

        @pl.loop(0, rpt // b)
        def _(i):
            pltpu.sync_copy(rows0, sum_sh.at[pl.ds(row0 + i * b, b)])

        plsc.subcore_barrier()

        tile_base = (c * NS + s) * ept

        def load_idx(slot, g, sem):
            base = tile_base + g * b
            pltpu.async_copy(src_hbm.at[pl.ds(base, b)], sidx.at[slot, 0], sem)
            pltpu.async_copy(dst_hbm.at[pl.ds(base, b)], didx.at[slot, 0], sem)

        def wait_idx(slot, sem):
            pltpu.make_async_copy(src_hbm.at[pl.ds(0, b)],
                                  sidx.at[slot, 0], sem).wait()
            pltpu.make_async_copy(dst_hbm.at[pl.ds(0, b)],
                                  didx.at[slot, 0], sem).wait()

        for j in range(min(5, nchunk)):
            load_idx(j, j, isem[j])
        wait_idx(0, isem[0])
        pltpu.async_copy(x_hbm.at[sidx.at[0, 0]], rows0, gsem0)
        if nchunk > 1:
            wait_idx(1, isem[1])
            pltpu.async_copy(x_hbm.at[sidx.at[1, 0]], rows1, gsem1)

        @pl.loop(0, (nchunk + 6) // 6)
        def _(t):
            for r in range(6):
                g = t * 6 + r
                r3 = r % 3
                r6 = r

                @pl.when(g < nchunk)
                def _(g=g, r3=r3, r6=r6):
                    pltpu.make_async_copy(x_hbm.at[sidx.at[r6, 0]],
                                          rows[r3], gsem[r3]).wait()
                    pltpu.async_copy(rows[r3], sum_sh.at[didx.at[r6, 0]],
                                     ssem[r3], add=True)

                    @pl.loop(0, b // CL)
                    def _(k):
                        dvec = didx[r6, 0, pl.ds(k * CL, CL)]
                        run, last = plsc.scan_count(dvec)
                        plsc.addupdate_scatter(hist, [dvec],
                                               run.astype(jnp.float32),
                                               mask=last)

                q3 = (r + 2) % 3
                q6 = (r + 5) % 6

                @pl.when(jnp.logical_and(g >= 1, g <= nchunk))
                def _(g=g, q3=q3, q6=q6):
                    pltpu.make_async_copy(rows[q3],
                                          sum_sh.at[didx.at[q6, 0]],
                                          ssem[q3]).wait()

                @pl.when(g + 5 < nchunk)
                def _(g=g, r6=r6):
                    load_idx((r6 + 5) % 6, g + 5, isem[(r6 + 5) % 6])

                @pl.when(g + 2 < nchunk)
                def _(g=g, r3=r3, r6=r6):
                    wait_idx((r6 + 2) % 6, isem[(r6 + 2) % 6])
                    pltpu.async_copy(x_hbm.at[sidx.at[(r6 + 2) % 6, 0]],
                                     rows[(r3 + 2) % 3], gsem[(r3 + 2) % 3])

        pltpu.sync_copy(hist, cnt_out.at[c].at[s])

        plsc.subcore_barrier()
        pltpu.sync_copy(sum_sh.at[pl.ds(row0, rpt)],
                        sum_out.at[c].at[pl.ds(row0, rpt)])

    return sc_agg(src, dst, x)


def _finish_body(sum_ref, inv_ref, x_ref, wl_ref, wr_ref, bl_ref, o_ref):
    mean = (sum_ref[0] + sum_ref[1]) * inv_ref[...]
    acc = jnp.dot(mean, wl_ref[...], preferred_element_type=jnp.float32)
    acc = acc + jnp.dot(x_ref[...], wr_ref[...], preferred_element_type=jnp.float32)
    acc = acc + bl_ref[...]
    o_ref[...] = jnp.maximum(acc, 0.0)


def _tc_finish(x, sum_p, inv_cnt, wl_t, wr_t, bl):
    n, d = x.shape
    bt = 2000 if n % 2000 == 0 else n
    grid = (n // bt,)
    return pl.pallas_call(
        _finish_body,
        grid=grid,
        in_specs=[
            pl.BlockSpec((NC, bt, d), lambda i: (0, i, 0)),
            pl.BlockSpec((bt, 1), lambda i: (i, 0)),
            pl.BlockSpec((bt, d), lambda i: (i, 0)),
            pl.BlockSpec((d, d), lambda i: (0, 0)),
            pl.BlockSpec((d, d), lambda i: (0, 0)),
            pl.BlockSpec((1, d), lambda i: (0, 0)),
        ],
        out_specs=pl.BlockSpec((bt, d), lambda i: (i, 0)),
        out_shape=jax.ShapeDtypeStruct((n, d), jnp.float32),
    )(sum_p, inv_cnt, x, wl_t, wr_t, bl)


def kernel(x, edge_index, W_l, b_l, W_r):
    n = x.shape[0]
    src = edge_index[0].astype(jnp.int32)
    dst = edge_index[1].astype(jnp.int32)
    sum_p, cnt_p = _sc_aggregate(src, dst, x)
    cnt = jnp.sum(cnt_p, axis=(0, 1))[:n]
    inv_cnt = (1.0 / jnp.maximum(cnt, 1.0)).reshape(n, 1)
    return _tc_finish(x, sum_p, inv_cnt, W_l.T, W_r.T, b_l.reshape(1, -1))


# speedup vs baseline: 15.1958x; 1.0016x over previous
import dataclasses
import functools

import jax
import jax.numpy as jnp
from jax import lax
from jax.experimental import pallas as pl
from jax.experimental.pallas import tpu as pltpu
from jax.experimental.pallas import tpu_sc as plsc

NC = 2
NS = 16
CL = 16


def _pick_chunk(ept: int) -> int:
    for b in range(128, 15, -16):
        if ept % b == 0:
            return b
    raise ValueError(f"per-tile edge count {ept} has no mult-of-16 divisor <= 128")


def _sc_aggregate(src, dst, x):
    n, d = x.shape
    e = src.shape[0]
    assert e % (NC * NS) == 0 and d % CL == 0
    ept = e // (NC * NS)
    b = _pick_chunk(ept)
    nchunk = ept // b
    npad = -(-n // (NS * CL)) * NS * CL
    rpt = npad // NS
    assert rpt % b == 0 and rpt % CL == 0

    mesh = plsc.VectorSubcoreMesh(core_axis_name="c", subcore_axis_name="s")
    cp = pltpu.CompilerParams()
    if "needs_layout_passes" in pltpu.CompilerParams.__dataclass_fields__:
        cp = dataclasses.replace(cp, needs_layout_passes=False)

    @functools.partial(
        pl.kernel,
        compiler_params=cp,
        out_type=(
            jax.ShapeDtypeStruct((NC, npad, d), jnp.float32),
            jax.ShapeDtypeStruct((NC, NS, npad), jnp.float32),
        ),
        mesh=mesh,
        scratch_types=[
            pltpu.VMEM_SHARED((npad, d), jnp.float32),
            pltpu.VMEM((6, 1, b), jnp.int32),
            pltpu.VMEM((6, 1, b), jnp.int32),
            pltpu.VMEM((b, d), jnp.float32),
            pltpu.VMEM((b, d), jnp.float32),
            pltpu.VMEM((b, d), jnp.float32),
            pltpu.VMEM((npad,), jnp.float32),
            pltpu.SemaphoreType.DMA,
            pltpu.SemaphoreType.DMA,
            pltpu.SemaphoreType.DMA,
            pltpu.SemaphoreType.DMA,
            pltpu.SemaphoreType.DMA,
            pltpu.SemaphoreType.DMA,
            pltpu.SemaphoreType.DMA,
            pltpu.SemaphoreType.DMA,
            pltpu.SemaphoreType.DMA,
            pltpu.SemaphoreType.DMA,
            pltpu.SemaphoreType.DMA,
            pltpu.SemaphoreType.DMA,
        ],
    )
    def sc_agg(src_hbm, dst_hbm, x_hbm, sum_out, cnt_out,
               sum_sh, sidx, didx, rows0, rows1, rows2, hist,
               gsem0, gsem1, gsem2, ssem0, ssem1, ssem2,
               isem0, isem1, isem2, isem3, isem4, isem5):
        c = lax.axis_index("c")
        s = lax.axis_index("s")
        rows = (rows0, rows1, rows2)
        gsem = (gsem0, gsem1, gsem2)
        ssem = (ssem0, ssem1, ssem2)
        isem = (isem0, isem1, isem2, isem3, isem4, isem5)

        @pl.loop(0, b)
        def _(r):
            @pl.loop(0, d // CL)
            def _(k):
                rows0[r, pl.ds(k * CL, CL)] = jnp.zeros((CL,), jnp.float32)

        @pl.loop(0, npad // CL)
        def _(i):
            hist[pl.ds(i * CL, CL)] = jnp.zeros((CL,), jnp.float32)

        row0 = s * rpt

        @pl.loop(0, rpt // b)
        def _(i):
            pltpu.sync_copy(rows0, sum_sh.at[pl.ds(row0 + i * b, b)])

        plsc.subcore_barrier()

        tile_base = (c * NS + s) * ept

        def load_idx(slot, g, sem):
            base = tile_base + g * b
            pltpu.async_copy(src_hbm.at[pl.ds(base, b)], sidx.at[slot, 0], sem)
            pltpu.async_copy(dst_hbm.at[pl.ds(base, b)], didx.at[slot, 0], sem)

        def wait_idx(slot, sem):
            pltpu.make_async_copy(src_hbm.at[pl.ds(0, b)],
                                  sidx.at[slot, 0], sem).wait()
            pltpu.make_async_copy(dst_hbm.at[pl.ds(0, b)],
                                  didx.at[slot, 0], sem).wait()

        for j in range(min(5, nchunk)):
            load_idx(j, j, isem[j])
        wait_idx(0, isem[0])
        pltpu.async_copy(x_hbm.at[sidx.at[0, 0]], rows0, gsem0)
        if nchunk > 1:
            wait_idx(1, isem[1])
            pltpu.async_copy(x_hbm.at[sidx.at[1, 0]], rows1, gsem1)

        @pl.loop(0, (nchunk + 6) // 6)
        def _(t):
            for r in range(6):
                g = t * 6 + r
                r3 = r % 3
                r6 = r

                q3 = (r + 2) % 3
                q6 = (r + 5) % 6

                @pl.when(jnp.logical_and(g >= 1, g <= nchunk))
                def _(g=g, q3=q3, q6=q6):
                    pltpu.make_async_copy(rows[q3],
                                          sum_sh.at[didx.at[q6, 0]],
                                          ssem[q3]).wait()

                @pl.when(g < nchunk)
                def _(g=g, r3=r3, r6=r6):
                    pltpu.make_async_copy(x_hbm.at[sidx.at[r6, 0]],
                                          rows[r3], gsem[r3]).wait()
                    pltpu.async_copy(rows[r3], sum_sh.at[didx.at[r6, 0]],
                                     ssem[r3], add=True)

                    @pl.loop(0, b // CL)
                    def _(k):
                        dvec = didx[r6, 0, pl.ds(k * CL, CL)]
                        run, last = plsc.scan_count(dvec)
                        plsc.addupdate_scatter(hist, [dvec],
                                               run.astype(jnp.float32),
                                               mask=last)

                @pl.when(g + 5 < nchunk)
                def _(g=g, r6=r6):
                    load_idx((r6 + 5) % 6, g + 5, isem[(r6 + 5) % 6])

                @pl.when(g + 2 < nchunk)
                def _(g=g, r3=r3, r6=r6):
                    wait_idx((r6 + 2) % 6, isem[(r6 + 2) % 6])
                    pltpu.async_copy(x_hbm.at[sidx.at[(r6 + 2) % 6, 0]],
                                     rows[(r3 + 2) % 3], gsem[(r3 + 2) % 3])

        pltpu.sync_copy(hist, cnt_out.at[c].at[s])

        plsc.subcore_barrier()
        pltpu.sync_copy(sum_sh.at[pl.ds(row0, rpt)],
                        sum_out.at[c].at[pl.ds(row0, rpt)])

    return sc_agg(src, dst, x)


def _finish_body(sum_ref, inv_ref, x_ref, wl_ref, wr_ref, bl_ref, o_ref):
    mean = (sum_ref[0] + sum_ref[1]) * inv_ref[...]
    acc = jnp.dot(mean, wl_ref[...], preferred_element_type=jnp.float32)
    acc = acc + jnp.dot(x_ref[...], wr_ref[...], preferred_element_type=jnp.float32)
    acc = acc + bl_ref[...]
    o_ref[...] = jnp.maximum(acc, 0.0)


def _tc_finish(x, sum_p, inv_cnt, wl_t, wr_t, bl):
    n, d = x.shape
    bt = 2000 if n % 2000 == 0 else n
    grid = (n // bt,)
    return pl.pallas_call(
        _finish_body,
        grid=grid,
        in_specs=[
            pl.BlockSpec((NC, bt, d), lambda i: (0, i, 0)),
            pl.BlockSpec((bt, 1), lambda i: (i, 0)),
            pl.BlockSpec((bt, d), lambda i: (i, 0)),
            pl.BlockSpec((d, d), lambda i: (0, 0)),
            pl.BlockSpec((d, d), lambda i: (0, 0)),
            pl.BlockSpec((1, d), lambda i: (0, 0)),
        ],
        out_specs=pl.BlockSpec((bt, d), lambda i: (i, 0)),
        out_shape=jax.ShapeDtypeStruct((n, d), jnp.float32),
    )(sum_p, inv_cnt, x, wl_t, wr_t, bl)


def kernel(x, edge_index, W_l, b_l, W_r):
    n = x.shape[0]
    src = edge_index[0].astype(jnp.int32)
    dst = edge_index[1].astype(jnp.int32)
    sum_p, cnt_p = _sc_aggregate(src, dst, x)
    cnt = jnp.sum(cnt_p, axis=(0, 1))[:n]
    inv_cnt = (1.0 / jnp.maximum(cnt, 1.0)).reshape(n, 1)
    return _tc_finish(x, sum_p, inv_cnt, W_l.T, W_r.T, b_l.reshape(1, -1))


# speedup vs baseline: 15.3087x; 1.0074x over previous
import dataclasses
import functools

import jax
import jax.numpy as jnp
from jax import lax
from jax.experimental import pallas as pl
from jax.experimental.pallas import tpu as pltpu
from jax.experimental.pallas import tpu_sc as plsc

NC = 2
NS = 16
CL = 16


def _pick_chunk(ept: int) -> int:
    for b in range(128, 15, -16):
        if ept % b == 0:
            return b
    raise ValueError(f"per-tile edge count {ept} has no mult-of-16 divisor <= 128")


def _sc_aggregate(src, dst, x):
    n, d = x.shape
    e = src.shape[0]
    assert e % (NC * NS) == 0 and d % CL == 0
    ept = e // (NC * NS)
    b = _pick_chunk(ept)
    nchunk = ept // b
    npad = -(-n // (NS * CL)) * NS * CL
    rpt = npad // NS
    assert rpt % b == 0 and rpt % CL == 0

    mesh = plsc.VectorSubcoreMesh(core_axis_name="c", subcore_axis_name="s")
    cp = pltpu.CompilerParams()
    if "needs_layout_passes" in pltpu.CompilerParams.__dataclass_fields__:
        cp = dataclasses.replace(cp, needs_layout_passes=False)

    @functools.partial(
        pl.kernel,
        compiler_params=cp,
        out_type=(
            jax.ShapeDtypeStruct((NC, npad, d), jnp.float32),
            jax.ShapeDtypeStruct((NC, NS, npad), jnp.float32),
        ),
        mesh=mesh,
        scratch_types=[
            pltpu.VMEM_SHARED((npad, d), jnp.float32),
            pltpu.VMEM((6, 1, b), jnp.int32),
            pltpu.VMEM((6, 1, b), jnp.int32),
            pltpu.VMEM((b, d), jnp.float32),
            pltpu.VMEM((b, d), jnp.float32),
            pltpu.VMEM((b, d), jnp.float32),
            pltpu.VMEM((npad,), jnp.float32),
            pltpu.SemaphoreType.DMA,
            pltpu.SemaphoreType.DMA,
            pltpu.SemaphoreType.DMA,
            pltpu.SemaphoreType.DMA,
            pltpu.SemaphoreType.DMA,
            pltpu.SemaphoreType.DMA,
            pltpu.SemaphoreType.DMA,
            pltpu.SemaphoreType.DMA,
            pltpu.SemaphoreType.DMA,
            pltpu.SemaphoreType.DMA,
            pltpu.SemaphoreType.DMA,
            pltpu.SemaphoreType.DMA,
            pltpu.SemaphoreType.DMA,
        ],
    )
    def sc_agg(src_hbm, dst_hbm, x_hbm, sum_out, cnt_out,
               sum_sh, sidx, didx, rows0, rows1, rows2, hist,
               gsem0, gsem1, gsem2, ssem0, ssem1, ssem2,
               isem0, isem1, isem2, isem3, isem4, isem5, zsem):
        c = lax.axis_index("c")
        s = lax.axis_index("s")
        rows = (rows0, rows1, rows2)
        gsem = (gsem0, gsem1, gsem2)
        ssem = (ssem0, ssem1, ssem2)
        isem = (isem0, isem1, isem2, isem3, isem4, isem5)

        @pl.loop(0, b)
        def _(r):
            @pl.loop(0, d // CL)
            def _(k):
                rows0[r, pl.ds(k * CL, CL)] = jnp.zeros((CL,), jnp.float32)

        @pl.loop(0, npad // CL)
        def _(i):
            hist[pl.ds(i * CL, CL)] = jnp.zeros((CL,), jnp.float32)

        row0 = s * rpt

        @pl.loop(0, rpt // b)
        def _(i):
            pltpu.async_copy(rows0, sum_sh.at[pl.ds(row0 + i * b, b)], zsem)

        tile_base = (c * NS + s) * ept

        def load_idx(slot, g, sem):
            base = tile_base + g * b
            pltpu.async_copy(src_hbm.at[pl.ds(base, b)], sidx.at[slot, 0], sem)
            pltpu.async_copy(dst_hbm.at[pl.ds(base, b)], didx.at[slot, 0], sem)

        def wait_idx(slot, sem):
            pltpu.make_async_copy(src_hbm.at[pl.ds(0, b)],
                                  sidx.at[slot, 0], sem).wait()
            pltpu.make_async_copy(dst_hbm.at[pl.ds(0, b)],
                                  didx.at[slot, 0], sem).wait()

        for j in range(min(5, nchunk)):
            load_idx(j, j, isem[j])

        @pl.loop(0, rpt // b)
        def _(i):
            pltpu.make_async_copy(rows0, sum_sh.at[pl.ds(row0, b)],
                                  zsem).wait()

        wait_idx(0, isem[0])
        pltpu.async_copy(x_hbm.at[sidx.at[0, 0]], rows0, gsem0)
        if nchunk > 1:
            wait_idx(1, isem[1])
            pltpu.async_copy(x_hbm.at[sidx.at[1, 0]], rows1, gsem1)
        plsc.subcore_barrier()

        @pl.loop(0, (nchunk + 6) // 6)
        def _(t):
            for r in range(6):
                g = t * 6 + r
                r3 = r % 3
                r6 = r

                q3 = (r + 2) % 3
                q6 = (r + 5) % 6

                @pl.when(jnp.logical_and(g >= 1, g <= nchunk))
                def _(g=g, q3=q3, q6=q6):
                    pltpu.make_async_copy(rows[q3],
                                          sum_sh.at[didx.at[q6, 0]],
                                          ssem[q3]).wait()

                @pl.when(g < nchunk)
                def _(g=g, r3=r3, r6=r6):
                    pltpu.make_async_copy(x_hbm.at[sidx.at[r6, 0]],
                                          rows[r3], gsem[r3]).wait()
                    pltpu.async_copy(rows[r3], sum_sh.at[didx.at[r6, 0]],
                                     ssem[r3], add=True)

                    @pl.loop(0, b // CL)
                    def _(k):
                        dvec = didx[r6, 0, pl.ds(k * CL, CL)]
                        run, last = plsc.scan_count(dvec)
                        plsc.addupdate_scatter(hist, [dvec],
                                               run.astype(jnp.float32),
                                               mask=last)

                @pl.when(g + 5 < nchunk)
                def _(g=g, r6=r6):
                    load_idx((r6 + 5) % 6, g + 5, isem[(r6 + 5) % 6])

                @pl.when(g + 2 < nchunk)
                def _(g=g, r3=r3, r6=r6):
                    wait_idx((r6 + 2) % 6, isem[(r6 + 2) % 6])
                    pltpu.async_copy(x_hbm.at[sidx.at[(r6 + 2) % 6, 0]],
                                     rows[(r3 + 2) % 3], gsem[(r3 + 2) % 3])

        pltpu.sync_copy(hist, cnt_out.at[c].at[s])

        plsc.subcore_barrier()
        pltpu.sync_copy(sum_sh.at[pl.ds(row0, rpt)],
                        sum_out.at[c].at[pl.ds(row0, rpt)])

    return sc_agg(src, dst, x)


def _finish_body(sum_ref, inv_ref, x_ref, wl_ref, wr_ref, bl_ref, o_ref):
    mean = (sum_ref[0] + sum_ref[1]) * inv_ref[...]
    acc = jnp.dot(mean, wl_ref[...], preferred_element_type=jnp.float32)
    acc = acc + jnp.dot(x_ref[...], wr_ref[...], preferred_element_type=jnp.float32)
    acc = acc + bl_ref[...]
    o_ref[...] = jnp.maximum(acc, 0.0)


def _tc_finish(x, sum_p, inv_cnt, wl_t, wr_t, bl):
    n, d = x.shape
    bt = 2000 if n % 2000 == 0 else n
    grid = (n // bt,)
    return pl.pallas_call(
        _finish_body,
        grid=grid,
        in_specs=[
            pl.BlockSpec((NC, bt, d), lambda i: (0, i, 0)),
            pl.BlockSpec((bt, 1), lambda i: (i, 0)),
            pl.BlockSpec((bt, d), lambda i: (i, 0)),
            pl.BlockSpec((d, d), lambda i: (0, 0)),
            pl.BlockSpec((d, d), lambda i: (0, 0)),
            pl.BlockSpec((1, d), lambda i: (0, 0)),
        ],
        out_specs=pl.BlockSpec((bt, d), lambda i: (i, 0)),
        out_shape=jax.ShapeDtypeStruct((n, d), jnp.float32),
    )(sum_p, inv_cnt, x, wl_t, wr_t, bl)


def kernel(x, edge_index, W_l, b_l, W_r):
    n = x.shape[0]
    src = edge_index[0].astype(jnp.int32)
    dst = edge_index[1].astype(jnp.int32)
    sum_p, cnt_p = _sc_aggregate(src, dst, x)
    cnt = jnp.sum(cnt_p, axis=(0, 1))[:n]
    inv_cnt = (1.0 / jnp.maximum(cnt, 1.0)).reshape(n, 1)
    return _tc_finish(x, sum_p, inv_cnt, W_l.T, W_r.T, b_l.reshape(1, -1))


# speedup vs baseline: 15.3219x; 1.0009x over previous
import dataclasses
import functools

import jax
import jax.numpy as jnp
from jax import lax
from jax.experimental import pallas as pl
from jax.experimental.pallas import tpu as pltpu
from jax.experimental.pallas import tpu_sc as plsc

NC = 2
NS = 16
CL = 16


def _pick_chunk(ept: int) -> int:
    for b in range(128, 15, -16):
        if ept % b == 0:
            return b
    raise ValueError(f"per-tile edge count {ept} has no mult-of-16 divisor <= 128")


def _sc_aggregate(src, dst, x):
    n, d = x.shape
    e = src.shape[0]
    assert e % (NC * NS) == 0 and d % CL == 0
    ept = e // (NC * NS)
    b = _pick_chunk(ept)
    nchunk = ept // b
    npad = -(-n // (NS * CL)) * NS * CL
    rpt = npad // NS
    assert rpt % b == 0 and rpt % CL == 0

    mesh = plsc.VectorSubcoreMesh(core_axis_name="c", subcore_axis_name="s")
    cp = pltpu.CompilerParams()
    if "needs_layout_passes" in pltpu.CompilerParams.__dataclass_fields__:
        cp = dataclasses.replace(cp, needs_layout_passes=False)

    @functools.partial(
        pl.kernel,
        compiler_params=cp,
        out_type=(
            jax.ShapeDtypeStruct((NC, npad, d), jnp.float32),
            jax.ShapeDtypeStruct((NC, NS, npad), jnp.float32),
        ),
        mesh=mesh,
        scratch_types=[
            pltpu.VMEM_SHARED((npad, d), jnp.float32),
            pltpu.VMEM((6, 1, b), jnp.int32),
            pltpu.VMEM((6, 1, b), jnp.int32),
            pltpu.VMEM((b, d), jnp.float32),
            pltpu.VMEM((b, d), jnp.float32),
            pltpu.VMEM((b, d), jnp.float32),
            pltpu.VMEM((npad,), jnp.float32),
            pltpu.SemaphoreType.DMA,
            pltpu.SemaphoreType.DMA,
            pltpu.SemaphoreType.DMA,
            pltpu.SemaphoreType.DMA,
            pltpu.SemaphoreType.DMA,
            pltpu.SemaphoreType.DMA,
            pltpu.SemaphoreType.DMA,
            pltpu.SemaphoreType.DMA,
            pltpu.SemaphoreType.DMA,
            pltpu.SemaphoreType.DMA,
            pltpu.SemaphoreType.DMA,
            pltpu.SemaphoreType.DMA,
            pltpu.SemaphoreType.DMA,
        ],
    )
    def sc_agg(src_hbm, dst_hbm, x_hbm, sum_out, cnt_out,
               sum_sh, sidx, didx, rows0, rows1, rows2, hist,
               gsem0, gsem1, gsem2, ssem0, ssem1, ssem2,
               isem0, isem1, isem2, isem3, isem4, isem5, zsem):
        c = lax.axis_index("c")
        s = lax.axis_index("s")
        rows = (rows0, rows1, rows2)
        gsem = (gsem0, gsem1, gsem2)
        ssem = (ssem0, ssem1, ssem2)
        isem = (isem0, isem1, isem2, isem3, isem4, isem5)

        @pl.loop(0, b)
        def _(r):
            @pl.loop(0, d // CL)
            def _(k):
                rows0[r, pl.ds(k * CL, CL)] = jnp.zeros((CL,), jnp.float32)

        @pl.loop(0, npad // CL)
        def _(i):
            hist[pl.ds(i * CL, CL)] = jnp.zeros((CL,), jnp.float32)

        row0 = s * rpt

        @pl.loop(0, rpt // b)
        def _(i):
            pltpu.async_copy(rows0, sum_sh.at[pl.ds(row0 + i * b, b)], zsem)

        tile_base = (c * NS + s) * ept

        def load_idx(slot, g, sem):
            base = tile_base + g * b
            pltpu.async_copy(src_hbm.at[pl.ds(base, b)], sidx.at[slot, 0], sem)
            pltpu.async_copy(dst_hbm.at[pl.ds(base, b)], didx.at[slot, 0], sem)

        def wait_idx(slot, sem):
            pltpu.make_async_copy(src_hbm.at[pl.ds(0, b)],
                                  sidx.at[slot, 0], sem).wait()
            pltpu.make_async_copy(dst_hbm.at[pl.ds(0, b)],
                                  didx.at[slot, 0], sem).wait()

        for j in range(min(5, nchunk)):
            load_idx(j, j, isem[j])

        @pl.loop(0, rpt // b)
        def _(i):
            pltpu.make_async_copy(rows0, sum_sh.at[pl.ds(row0, b)],
                                  zsem).wait()

        wait_idx(0, isem[0])
        pltpu.async_copy(x_hbm.at[sidx.at[0, 0]], rows0, gsem0)
        if nchunk > 1:
            wait_idx(1, isem[1])
            pltpu.async_copy(x_hbm.at[sidx.at[1, 0]], rows1, gsem1)
        plsc.subcore_barrier()

        @pl.loop(0, (nchunk + 6) // 6)
        def _(t):
            for r in range(6):
                g = t * 6 + r
                r3 = r % 3
                r6 = r

                q3 = (r + 2) % 3
                q6 = (r + 5) % 6

                @pl.when(jnp.logical_and(g >= 1, g <= nchunk))
                def _(g=g, q3=q3, q6=q6):
                    pltpu.make_async_copy(rows[q3],
                                          sum_sh.at[didx.at[q6, 0]],
                                          ssem[q3]).wait()

                @pl.when(g < nchunk)
                def _(g=g, r3=r3, r6=r6):
                    pltpu.make_async_copy(x_hbm.at[sidx.at[r6, 0]],
                                          rows[r3], gsem[r3]).wait()
                    pltpu.async_copy(rows[r3], sum_sh.at[didx.at[r6, 0]],
                                     ssem[r3], add=True)

                    @pl.loop(0, b // CL)
                    def _(k):
                        dvec = didx[r6, 0, pl.ds(k * CL, CL)]
                        run, last = plsc.scan_count(dvec)
                        plsc.addupdate_scatter(hist, [dvec],
                                               run.astype(jnp.float32),
                                               mask=last)

                @pl.when(g + 5 < nchunk)
                def _(g=g, r6=r6):
                    load_idx((r6 + 5) % 6, g + 5, isem[(r6 + 5) % 6])

                @pl.when(g + 2 < nchunk)
                def _(g=g, r3=r3, r6=r6):
                    wait_idx((r6 + 2) % 6, isem[(r6 + 2) % 6])
                    pltpu.async_copy(x_hbm.at[sidx.at[(r6 + 2) % 6, 0]],
                                     rows[(r3 + 2) % 3], gsem[(r3 + 2) % 3])

        pltpu.async_copy(hist, cnt_out.at[c].at[s], zsem)

        plsc.subcore_barrier()
        pltpu.sync_copy(sum_sh.at[pl.ds(row0, rpt)],
                        sum_out.at[c].at[pl.ds(row0, rpt)])
        pltpu.make_async_copy(hist, cnt_out.at[c].at[s], zsem).wait()

    return sc_agg(src, dst, x)


def _finish_body(sum_ref, inv_ref, x_ref, wl_ref, wr_ref, bl_ref, o_ref):
    mean = (sum_ref[0] + sum_ref[1]) * inv_ref[...]
    acc = jnp.dot(mean, wl_ref[...], preferred_element_type=jnp.float32)
    acc = acc + jnp.dot(x_ref[...], wr_ref[...], preferred_element_type=jnp.float32)
    acc = acc + bl_ref[...]
    o_ref[...] = jnp.maximum(acc, 0.0)


def _tc_finish(x, sum_p, inv_cnt, wl_t, wr_t, bl):
    n, d = x.shape
    bt = 2000 if n % 2000 == 0 else n
    grid = (n // bt,)
    return pl.pallas_call(
        _finish_body,
        grid=grid,
        in_specs=[
            pl.BlockSpec((NC, bt, d), lambda i: (0, i, 0)),
            pl.BlockSpec((bt, 1), lambda i: (i, 0)),
            pl.BlockSpec((bt, d), lambda i: (i, 0)),
            pl.BlockSpec((d, d), lambda i: (0, 0)),
            pl.BlockSpec((d, d), lambda i: (0, 0)),
            pl.BlockSpec((1, d), lambda i: (0, 0)),
        ],
        out_specs=pl.BlockSpec((bt, d), lambda i: (i, 0)),
        out_shape=jax.ShapeDtypeStruct((n, d), jnp.float32),
    )(sum_p, inv_cnt, x, wl_t, wr_t, bl)


def kernel(x, edge_index, W_l, b_l, W_r):
    n = x.shape[0]
    src = edge_index[0].astype(jnp.int32)
    dst = edge_index[1].astype(jnp.int32)
    sum_p, cnt_p = _sc_aggregate(src, dst, x)
    cnt = jnp.sum(cnt_p, axis=(0, 1))[:n]
    inv_cnt = (1.0 / jnp.maximum(cnt, 1.0)).reshape(n, 1)
    return _tc_finish(x, sum_p, inv_cnt, W_l.T, W_r.T, b_l.reshape(1, -1))
